# Initial kernel scaffold; baseline (speedup 1.0000x reference)
import jax
import jax.numpy as jnp
from jax.experimental import pallas as pl


def kernel(x, edge_index, edge_attr, W1, b1, W2, b2, gamma, beta):
    raise NotImplementedError("write your pallas kernel here")


# speedup vs baseline: 4.6064x; 4.6064x over previous
import functools

import jax
import jax.numpy as jnp
from jax import lax
from jax.experimental import pallas as pl
from jax.experimental.pallas import tpu as pltpu
from jax.experimental.pallas import tpu_sc as plsc

_NC = 2
_NS = 16
_LANES = 16


def _sc_aggregate(x, src, dst, edge_attr):
    n, d = x.shape
    e = src.shape[0]
    nw = _NC * _NS
    assert e % nw == 0
    ew = e // nw
    c_main = 128
    n_chunk = ew // c_main
    c_tail = ew - n_chunk * c_main
    zrows = 64
    rows_sub = -(-n // (_NS * zrows)) * zrows
    n_pad = rows_sub * _NS
    assert d % _LANES == 0
    vecs_per_row = d // _LANES

    mesh = plsc.VectorSubcoreMesh(core_axis_name="c", subcore_axis_name="s")

    def body(x_hbm, src_hbm, dst_hbm, ea_hbm, part_hbm,
             srcv, dstv, xv, ev, srcv_t, dstv_t, xv_t, ev_t, zb, aggr, sem):
        cid = lax.axis_index("c")
        sid = lax.axis_index("s")
        wid = sid * _NC + cid

        zero16 = jnp.zeros((_LANES,), jnp.float32)

        def zrow(i, carry):
            for j in range(vecs_per_row):
                zb[i, pl.ds(j * _LANES, _LANES)] = zero16
            return carry

        lax.fori_loop(0, zrows, zrow, 0)
        row0 = sid * rows_sub
        for r in range(rows_sub // zrows):
            pltpu.sync_copy(zb, aggr.at[pl.ds(row0 + r * zrows, zrows), :])
        plsc.subcore_barrier()

        base0 = wid * ew

        def relu_rows(nrows, xbuf, ebuf):
            def rowfn(i, carry):
                for j in range(vecs_per_row):
                    sl = pl.ds(j * _LANES, _LANES)
                    ebuf[i, sl] = jnp.maximum(xbuf[i, sl] + ebuf[i, sl], 0.0)
                return carry
            lax.fori_loop(0, nrows, rowfn, 0)

        def chunk(t, carry):
            base = base0 + t * c_main
            pltpu.sync_copy(src_hbm.at[pl.ds(base, c_main)], srcv)
            pltpu.sync_copy(dst_hbm.at[pl.ds(base, c_main)], dstv)
            gat = pltpu.async_copy(x_hbm.at[srcv], xv, sem)
            pltpu.sync_copy(ea_hbm.at[pl.ds(base, c_main), :], ev)
            gat.wait()
            relu_rows(c_main, xv, ev)
            pltpu.sync_copy(ev, aggr.at[dstv], add=True)
            return carry

        lax.fori_loop(0, n_chunk, chunk, 0)

        if c_tail:
            base = base0 + n_chunk * c_main
            pltpu.sync_copy(src_hbm.at[pl.ds(base, c_tail)], srcv_t)
            pltpu.sync_copy(dst_hbm.at[pl.ds(base, c_tail)], dstv_t)
            gat = pltpu.async_copy(x_hbm.at[srcv_t], xv_t, sem)
            pltpu.sync_copy(ea_hbm.at[pl.ds(base, c_tail), :], ev_t)
            gat.wait()
            relu_rows(c_tail, xv_t, ev_t)
            pltpu.sync_copy(ev_t, aggr.at[dstv_t], add=True)

        plsc.subcore_barrier()
        pltpu.sync_copy(aggr.at[pl.ds(row0, rows_sub), :],
                        part_hbm.at[pl.ds(cid * n_pad + row0, rows_sub), :])

    run = pl.kernel(
        body,
        out_type=jax.ShapeDtypeStruct((_NC * n_pad, d), jnp.float32),
        mesh=mesh,
        scratch_types=[
            pltpu.VMEM((c_main,), jnp.int32),
            pltpu.VMEM((c_main,), jnp.int32),
            pltpu.VMEM((c_main, d), jnp.float32),
            pltpu.VMEM((c_main, d), jnp.float32),
            pltpu.VMEM((max(c_tail, 8),), jnp.int32),
            pltpu.VMEM((max(c_tail, 8),), jnp.int32),
            pltpu.VMEM((max(c_tail, 8), d), jnp.float32),
            pltpu.VMEM((max(c_tail, 8), d), jnp.float32),
            pltpu.VMEM((zrows, d), jnp.float32),
            pltpu.VMEM_SHARED((n_pad, d), jnp.float32),
            pltpu.SemaphoreType.DMA,
        ],
    )
    part = run(x, src, dst, edge_attr)
    return part[:n], part[n_pad:n_pad + n]


def _tc_body(x_ref, p0_ref, p1_ref, w1_ref, b1_ref, w2_ref, b2_ref,
             g_ref, bt_ref, out_ref):
    x = x_ref[...]
    z = x + p0_ref[...] + p1_ref[...]
    h = jnp.dot(z, w1_ref[...], preferred_element_type=jnp.float32) + b1_ref[...]
    h = jnp.maximum(h, 0.0)
    h = jnp.dot(h, w2_ref[...], preferred_element_type=jnp.float32) + b2_ref[...]
    r = x + jnp.maximum(h, 0.0)
    mean = jnp.mean(r, axis=1, keepdims=True)
    cen = r - mean
    var = jnp.mean(cen * cen, axis=1, keepdims=True)
    out_ref[...] = cen * lax.rsqrt(var + 1e-5) * g_ref[...] + bt_ref[...]


def _tc_mlp(x, p0, p1, W1, b1, W2, b2, gamma, beta):
    n, d = x.shape
    bn = 2000
    assert n % bn == 0
    grid = n // bn
    row_spec = pl.BlockSpec((bn, d), lambda i: (i, 0))
    full_spec = pl.BlockSpec((d, d), lambda i: (0, 0))
    vec_spec = pl.BlockSpec((1, d), lambda i: (0, 0))
    return pl.pallas_call(
        _tc_body,
        grid=(grid,),
        in_specs=[row_spec, row_spec, row_spec, full_spec, vec_spec,
                  full_spec, vec_spec, vec_spec, vec_spec],
        out_specs=row_spec,
        out_shape=jax.ShapeDtypeStruct((n, d), jnp.float32),
    )(x, p0, p1, W1, b1.reshape(1, d), W2, b2.reshape(1, d),
      gamma.reshape(1, d), beta.reshape(1, d))


def kernel(x, edge_index, edge_attr, W1, b1, W2, b2, gamma, beta):
    ei = edge_index.astype(jnp.int32)
    p0, p1 = _sc_aggregate(x, ei[0], ei[1], edge_attr)
    return _tc_mlp(x, p0, p1, W1, b1, W2, b2, gamma, beta)
